# Initial kernel scaffold; baseline (speedup 1.0000x reference)
import jax
import jax.numpy as jnp
from jax.experimental import pallas as pl


def kernel(x, edge_index, e, w_k1, b_k1, root1, bias1, w_k2, b_k2, root2, bias2, dense_w, dense_b):
    raise NotImplementedError("write your pallas kernel here")


# speedup vs baseline: 1.6717x; 1.6717x over previous
import functools

import jax
import jax.numpy as jnp
from jax import lax
from jax.experimental import pallas as pl
from jax.experimental.pallas import tpu as pltpu
from jax.experimental.pallas import tpu_sc as plsc

N = 10000
E = 80000
F = 32
D = 16

NC, NS = 2, 16
NW = NC * NS
EP = 81920
EPW = EP // NW
ECH = EPW // 128
NP = 10240
RPT = NP // NS

_MESH = dict(core_axis_name="c", subcore_axis_name="s")


def _sc_gather(table, idx2):
    @functools.partial(
        pl.kernel,
        out_type=jax.ShapeDtypeStruct((EP, F), jnp.float32),
        mesh=plsc.VectorSubcoreMesh(**_MESH),
        scratch_types=[
            pltpu.VMEM((ECH, 128), jnp.int32),
            pltpu.VMEM((EPW, F), jnp.float32),
            pltpu.SemaphoreType.DMA,
        ],
        compiler_params=pltpu.CompilerParams(use_tc_tiling_on_sc=False),
    )
    def k(table_hbm, idx_hbm, out_hbm, idx_v, rows_v, sem):
        cid = lax.axis_index("c")
        sid = lax.axis_index("s")
        wid = sid * NC + cid
        pltpu.sync_copy(idx_hbm.at[wid], idx_v)
        copies = [
            pltpu.async_copy(table_hbm.at[idx_v.at[j]],
                             rows_v.at[pl.ds(j * 128, 128)], sem)
            for j in range(ECH)
        ]
        for c in copies:
            c.wait()
        pltpu.sync_copy(rows_v, out_hbm.at[pl.ds(wid * EPW, EPW)])

    return k(table, idx2)


def _sc_scatter(m, tgt2, zeros_np):
    @functools.partial(
        pl.kernel,
        out_type=jax.ShapeDtypeStruct((NC, NP, F), jnp.float32),
        mesh=plsc.VectorSubcoreMesh(**_MESH),
        scratch_types=[
            pltpu.VMEM((ECH, 128), jnp.int32),
            pltpu.VMEM((EPW, F), jnp.float32),
            pltpu.VMEM((RPT, F), jnp.float32),
            pltpu.VMEM_SHARED((NP, F), jnp.float32),
            pltpu.SemaphoreType.DMA,
        ],
        compiler_params=pltpu.CompilerParams(use_tc_tiling_on_sc=False),
    )
    def k(m_hbm, tgt_hbm, z_hbm, p_hbm, idx_v, m_v, stage_v, acc_sh, sem):
        cid = lax.axis_index("c")
        sid = lax.axis_index("s")
        wid = sid * NC + cid
        pltpu.sync_copy(z_hbm.at[pl.ds(sid * RPT, RPT)], stage_v)
        pltpu.sync_copy(stage_v, acc_sh.at[pl.ds(sid * RPT, RPT)])
        pltpu.sync_copy(tgt_hbm.at[wid], idx_v)
        pltpu.sync_copy(m_hbm.at[pl.ds(wid * EPW, EPW)], m_v)
        plsc.subcore_barrier()
        adds = [
            pltpu.async_copy(m_v.at[pl.ds(j * 128, 128)],
                             acc_sh.at[idx_v.at[j]], sem, add=True)
            for j in range(ECH)
        ]
        for c in adds:
            c.wait()
        plsc.subcore_barrier()
        pltpu.sync_copy(acc_sh.at[pl.ds(sid * RPT, RPT)], stage_v)
        pltpu.sync_copy(stage_v, p_hbm.at[cid, pl.ds(sid * RPT, RPT)])

    return k(m, tgt2, zeros_np)


def _tc_messages(e_p, msgs, Wt, Bm, xh_p, root, bias1r):
    GRID = 128
    TB = EP // GRID
    NB = NP // GRID

    def body(e_ref, mg_ref, wt_ref, bm_ref, x_ref, root_ref, b_ref,
             m_ref, r_ref):
        eb = e_ref[...]
        mb = mg_ref[...]
        z = jnp.concatenate([eb[:, d:d + 1] * mb for d in range(D)], axis=1)
        m = jax.lax.dot_general(z, wt_ref[...], (((1,), (0,)), ((), ())),
                                preferred_element_type=jnp.float32)
        m_ref[...] = m + mb @ bm_ref[...]
        r_ref[...] = x_ref[...] @ root_ref[...] + b_ref[...]

    return pl.pallas_call(
        body,
        grid=(GRID,),
        in_specs=[
            pl.BlockSpec((TB, D), lambda i: (i, 0)),
            pl.BlockSpec((TB, F), lambda i: (i, 0)),
            pl.BlockSpec((D * F, F), lambda i: (0, 0)),
            pl.BlockSpec((F, F), lambda i: (0, 0)),
            pl.BlockSpec((NB, F), lambda i: (i, 0)),
            pl.BlockSpec((F, F), lambda i: (0, 0)),
            pl.BlockSpec((1, F), lambda i: (0, 0)),
        ],
        out_specs=[
            pl.BlockSpec((TB, F), lambda i: (i, 0)),
            pl.BlockSpec((NB, F), lambda i: (i, 0)),
        ],
        out_shape=[
            jax.ShapeDtypeStruct((EP, F), jnp.float32),
            jax.ShapeDtypeStruct((NP, F), jnp.float32),
        ],
    )(e_p, msgs, Wt, Bm, xh_p, root, bias1r)


def _tc_relu3(p, r):
    def body(p_ref, r_ref, h_ref):
        h_ref[...] = jnp.maximum(p_ref[0] + p_ref[1] + r_ref[...], 0.0)

    return pl.pallas_call(
        body,
        out_shape=jax.ShapeDtypeStruct((NP, F), jnp.float32),
    )(p, r)


def _tc_final(pa, pb, r, dw, db):
    def body(a_ref, b_ref, r_ref, w_ref, db_ref, o_ref):
        h = jnp.maximum(a_ref[...] + b_ref[...] + r_ref[...], 0.0)
        pooled = jnp.sum(h, axis=0, keepdims=True)
        o_ref[...] = pooled @ w_ref[...] + db_ref[...]

    return pl.pallas_call(
        body,
        out_shape=jax.ShapeDtypeStruct((1, 1), jnp.float32),
    )(pa, pb, r, dw, db.reshape(1, 1))


def kernel(x, edge_index, e, w_k1, b_k1, root1, bias1,
           w_k2, b_k2, root2, bias2, dense_w, dense_b):
    src = edge_index[0]
    tgt = edge_index[1]
    src2 = jnp.concatenate(
        [src, jnp.zeros((EP - E,), jnp.int32)]).reshape(NW, ECH, 128)
    tgt2 = jnp.concatenate(
        [tgt, jnp.full((EP - E,), N, jnp.int32)]).reshape(NW, ECH, 128)
    e_p = jnp.concatenate([e, jnp.zeros((EP - E, D), jnp.float32)])
    x_p = jnp.concatenate([x, jnp.zeros((NP - N, F), jnp.float32)])
    Wt1 = w_k1.reshape(D * F, F)
    Bm1 = b_k1.reshape(F, F)
    Wt2 = w_k2.reshape(D * F, F)
    Bm2 = b_k2.reshape(F, F)
    zeros_np = jnp.zeros((NP, F), jnp.float32)

    msgs1 = _sc_gather(x, src2)
    m1, r1 = _tc_messages(e_p, msgs1, Wt1, Bm1, x_p, root1,
                          bias1.reshape(1, F))
    p1 = _sc_scatter(m1, tgt2, zeros_np)
    h1 = _tc_relu3(p1, r1)
    msgs2 = _sc_gather(h1, src2)
    m2, r2 = _tc_messages(e_p, msgs2, Wt2, Bm2, h1, root2,
                          bias2.reshape(1, F))
    p2 = _sc_scatter(m2, tgt2, zeros_np)
    return _tc_final(p2[0, :N], p2[1, :N], r2[:N], dense_w, dense_b)


# speedup vs baseline: 2.6138x; 1.5635x over previous
import functools

import jax
import jax.numpy as jnp
from jax import lax
from jax.experimental import pallas as pl
from jax.experimental.pallas import tpu as pltpu
from jax.experimental.pallas import tpu_sc as plsc

N = 10000
E = 80000
F = 32
D = 16

NC, NS = 2, 16
NW = NC * NS
EP = 81920
EPW = EP // NW
ECH = EPW // 128
NP = 10240
RPT = NP // NS

_MESH = dict(core_axis_name="c", subcore_axis_name="s")


def _sc_gather(table, idx2):
    @functools.partial(
        pl.kernel,
        out_type=jax.ShapeDtypeStruct((EP, F), jnp.float32),
        mesh=plsc.VectorSubcoreMesh(**_MESH),
        scratch_types=[
            pltpu.VMEM((ECH, 128), jnp.int32),
            pltpu.VMEM((EPW, F), jnp.float32),
            pltpu.SemaphoreType.DMA,
        ],
        compiler_params=pltpu.CompilerParams(use_tc_tiling_on_sc=False),
    )
    def k(table_hbm, idx_hbm, out_hbm, idx_v, rows_v, sem):
        cid = lax.axis_index("c")
        sid = lax.axis_index("s")
        wid = sid * NC + cid
        pltpu.sync_copy(idx_hbm.at[wid], idx_v)
        copies = [
            pltpu.async_copy(table_hbm.at[idx_v.at[j]],
                             rows_v.at[pl.ds(j * 128, 128)], sem)
            for j in range(ECH)
        ]
        for c in copies:
            c.wait()
        pltpu.sync_copy(rows_v, out_hbm.at[pl.ds(wid * EPW, EPW)])

    return k(table, idx2)


def _sc_scatter(m, tgt2, zeros_np):
    @functools.partial(
        pl.kernel,
        out_type=jax.ShapeDtypeStruct((NC, NP, F), jnp.float32),
        mesh=plsc.VectorSubcoreMesh(**_MESH),
        scratch_types=[
            pltpu.VMEM((ECH, 128), jnp.int32),
            pltpu.VMEM((EPW, F), jnp.float32),
            pltpu.VMEM((RPT, F), jnp.float32),
            pltpu.VMEM_SHARED((NP, F), jnp.float32),
            pltpu.SemaphoreType.DMA,
        ],
        compiler_params=pltpu.CompilerParams(use_tc_tiling_on_sc=False),
    )
    def k(m_hbm, tgt_hbm, z_hbm, p_hbm, idx_v, m_v, stage_v, acc_sh, sem):
        cid = lax.axis_index("c")
        sid = lax.axis_index("s")
        wid = sid * NC + cid
        pltpu.sync_copy(z_hbm.at[pl.ds(sid * RPT, RPT)], stage_v)
        pltpu.sync_copy(stage_v, acc_sh.at[pl.ds(sid * RPT, RPT)])
        pltpu.sync_copy(tgt_hbm.at[wid], idx_v)
        pltpu.sync_copy(m_hbm.at[pl.ds(wid * EPW, EPW)], m_v)
        plsc.subcore_barrier()
        adds = [
            pltpu.async_copy(m_v.at[pl.ds(j * 128, 128)],
                             acc_sh.at[idx_v.at[j]], sem, add=True)
            for j in range(ECH)
        ]
        for c in adds:
            c.wait()
        plsc.subcore_barrier()
        pltpu.sync_copy(acc_sh.at[pl.ds(sid * RPT, RPT)], stage_v)
        pltpu.sync_copy(stage_v, p_hbm.at[cid, pl.ds(sid * RPT, RPT)])

    return k(m, tgt2, zeros_np)


def _tc_messages(e_p, msgs, Wt, Bm, xh_p, root, bias1r):
    GRID = 128
    TB = EP // GRID
    NB = NP // GRID

    S = (jnp.arange(512)[None, :] // F == jnp.arange(D)[:, None]
         ).astype(jnp.bfloat16)
    T = (jnp.arange(512)[None, :] % F == jnp.arange(F)[:, None]
         ).astype(jnp.bfloat16)
    dn = (((1,), (0,)), ((), ()))

    def body(e_ref, mg_ref, s_ref, t_ref, wt_ref, bm_ref, x_ref, root_ref,
             b_ref, m_ref, r_ref):
        eb = e_ref[...].astype(jnp.bfloat16)
        mb = mg_ref[...]
        mb16 = mb.astype(jnp.bfloat16)
        e_rep = jax.lax.dot_general(eb, s_ref[...], dn,
                                    preferred_element_type=jnp.float32)
        m_rep = jax.lax.dot_general(mb16, t_ref[...], dn,
                                    preferred_element_type=jnp.float32)
        z = (e_rep * m_rep).astype(jnp.bfloat16)
        m = jax.lax.dot_general(z, wt_ref[...], dn,
                                preferred_element_type=jnp.float32)
        m_ref[...] = m + mb @ bm_ref[...]
        r_ref[...] = x_ref[...] @ root_ref[...] + b_ref[...]

    return pl.pallas_call(
        body,
        grid=(GRID,),
        in_specs=[
            pl.BlockSpec((TB, D), lambda i: (i, 0)),
            pl.BlockSpec((TB, F), lambda i: (i, 0)),
            pl.BlockSpec((D, D * F), lambda i: (0, 0)),
            pl.BlockSpec((F, D * F), lambda i: (0, 0)),
            pl.BlockSpec((D * F, F), lambda i: (0, 0)),
            pl.BlockSpec((F, F), lambda i: (0, 0)),
            pl.BlockSpec((NB, F), lambda i: (i, 0)),
            pl.BlockSpec((F, F), lambda i: (0, 0)),
            pl.BlockSpec((1, F), lambda i: (0, 0)),
        ],
        out_specs=[
            pl.BlockSpec((TB, F), lambda i: (i, 0)),
            pl.BlockSpec((NB, F), lambda i: (i, 0)),
        ],
        out_shape=[
            jax.ShapeDtypeStruct((EP, F), jnp.float32),
            jax.ShapeDtypeStruct((NP, F), jnp.float32),
        ],
    )(e_p, msgs, S, T, Wt.astype(jnp.bfloat16), Bm, xh_p, root, bias1r)


def _tc_relu3(p, r):
    def body(p_ref, r_ref, h_ref):
        h_ref[...] = jnp.maximum(p_ref[0] + p_ref[1] + r_ref[...], 0.0)

    return pl.pallas_call(
        body,
        out_shape=jax.ShapeDtypeStruct((NP, F), jnp.float32),
    )(p, r)


def _tc_final(pa, pb, r, dw, db):
    def body(a_ref, b_ref, r_ref, w_ref, db_ref, o_ref):
        h = jnp.maximum(a_ref[...] + b_ref[...] + r_ref[...], 0.0)
        pooled = jnp.sum(h, axis=0, keepdims=True)
        o_ref[...] = pooled @ w_ref[...] + db_ref[...]

    return pl.pallas_call(
        body,
        out_shape=jax.ShapeDtypeStruct((1, 1), jnp.float32),
    )(pa, pb, r, dw, db.reshape(1, 1))


def kernel(x, edge_index, e, w_k1, b_k1, root1, bias1,
           w_k2, b_k2, root2, bias2, dense_w, dense_b):
    src = edge_index[0]
    tgt = edge_index[1]
    src2 = jnp.concatenate(
        [src, jnp.zeros((EP - E,), jnp.int32)]).reshape(NW, ECH, 128)
    tgt2 = jnp.concatenate(
        [tgt, jnp.full((EP - E,), N, jnp.int32)]).reshape(NW, ECH, 128)
    e_p = jnp.concatenate([e, jnp.zeros((EP - E, D), jnp.float32)])
    x_p = jnp.concatenate([x, jnp.zeros((NP - N, F), jnp.float32)])
    Wt1 = w_k1.reshape(D * F, F)
    Bm1 = b_k1.reshape(F, F)
    Wt2 = w_k2.reshape(D * F, F)
    Bm2 = b_k2.reshape(F, F)
    zeros_np = jnp.zeros((NP, F), jnp.float32)

    msgs1 = _sc_gather(x, src2)
    m1, r1 = _tc_messages(e_p, msgs1, Wt1, Bm1, x_p, root1,
                          bias1.reshape(1, F))
    p1 = _sc_scatter(m1, tgt2, zeros_np)
    h1 = _tc_relu3(p1, r1)
    msgs2 = _sc_gather(h1, src2)
    m2, r2 = _tc_messages(e_p, msgs2, Wt2, Bm2, h1, root2,
                          bias2.reshape(1, F))
    p2 = _sc_scatter(m2, tgt2, zeros_np)
    return _tc_final(p2[0, :N], p2[1, :N], r2[:N], dense_w, dense_b)


# speedup vs baseline: 3.3541x; 1.2832x over previous
import functools

import jax
import jax.numpy as jnp
from jax import lax
from jax.experimental import pallas as pl
from jax.experimental.pallas import tpu as pltpu
from jax.experimental.pallas import tpu_sc as plsc

N = 10000
E = 80000
F = 32
D = 16

NC, NS = 2, 16
NW = NC * NS
EP = 81920
EPW = EP // NW
NCH = 20
NP = 10240
RPT = NP // NS
EQ = EP // 4

_MESH = dict(core_axis_name="c", subcore_axis_name="s")
_SC_PARAMS = pltpu.CompilerParams(use_tc_tiling_on_sc=False)


def _sc_gather(table, idx3):
    @functools.partial(
        pl.kernel,
        out_type=jax.ShapeDtypeStruct((EP, F), jnp.float32),
        mesh=plsc.VectorSubcoreMesh(**_MESH),
        scratch_types=[
            pltpu.VMEM((NCH, 128), jnp.int32),
            pltpu.VMEM((EPW, F), jnp.float32),
            pltpu.SemaphoreType.DMA,
        ],
        compiler_params=_SC_PARAMS,
    )
    def k(table_hbm, idx_hbm, out_hbm, idx_v, rows_v, sem):
        cid = lax.axis_index("c")
        sid = lax.axis_index("s")
        wid = sid * NC + cid
        pltpu.sync_copy(idx_hbm.at[wid], idx_v)
        copies = [
            pltpu.async_copy(table_hbm.at[idx_v.at[j]],
                             rows_v.at[pl.ds(j * 128, 128)], sem)
            for j in range(NCH)
        ]
        for c in copies:
            c.wait()
        pltpu.sync_copy(rows_v, out_hbm.at[pl.ds(wid * EPW, EPW)])

    return k(table, idx3)


def _sc_scatter(m, tgt3, zeros_np):
    @functools.partial(
        pl.kernel,
        out_type=jax.ShapeDtypeStruct((NC, NP, F), jnp.float32),
        mesh=plsc.VectorSubcoreMesh(**_MESH),
        scratch_types=[
            pltpu.VMEM((NCH, 128), jnp.int32),
            pltpu.VMEM((EPW, F), jnp.float32),
            pltpu.VMEM((RPT, F), jnp.float32),
            pltpu.VMEM_SHARED((NP, F), jnp.float32),
            pltpu.SemaphoreType.DMA,
        ],
        compiler_params=_SC_PARAMS,
    )
    def k(m_hbm, tgt_hbm, z_hbm, p_hbm, idx_v, m_v, stage_v, acc_sh, sem):
        cid = lax.axis_index("c")
        sid = lax.axis_index("s")
        wid = sid * NC + cid
        pltpu.sync_copy(z_hbm.at[pl.ds(sid * RPT, RPT)], stage_v)
        pltpu.sync_copy(stage_v, acc_sh.at[pl.ds(sid * RPT, RPT)])
        pltpu.sync_copy(tgt_hbm.at[wid], idx_v)
        pltpu.sync_copy(m_hbm.at[pl.ds(wid * EPW, EPW)], m_v)
        plsc.subcore_barrier()
        adds = [
            pltpu.async_copy(m_v.at[pl.ds(j * 128, 128)],
                             acc_sh.at[idx_v.at[j]], sem, add=True)
            for j in range(NCH)
        ]
        for c in adds:
            c.wait()
        plsc.subcore_barrier()
        pltpu.sync_copy(acc_sh.at[pl.ds(sid * RPT, RPT)], stage_v)
        pltpu.sync_copy(stage_v, p_hbm.at[cid, pl.ds(sid * RPT, RPT)])

    return k(m, tgt3, zeros_np)


def _tc_messages(e4, msgs_p, W4, B4, xh, root, bias1r):
    GRID = 128
    TQ = EQ // GRID
    NB = NP // GRID

    v = jnp.arange(4 * D * F)
    jj = v // (D * F)
    dd = (v % (D * F)) // F
    bb = v % F
    S4 = (jnp.arange(4 * D)[:, None] == (jj * D + dd)[None, :]
          ).astype(jnp.bfloat16)
    T4 = (jnp.arange(4 * F)[:, None] == (jj * F + bb)[None, :]
          ).astype(jnp.bfloat16)
    dn = (((1,), (0,)), ((), ()))

    def body(e_ref, mg_ref, s_ref, t_ref, w4_ref, b4_ref, x_ref, root_ref,
             b_ref, m_ref, r_ref):
        eb = e_ref[...].astype(jnp.bfloat16)
        mp = mg_ref[...]
        mp16 = mp.astype(jnp.bfloat16)
        e_rep = jax.lax.dot_general(eb, s_ref[...], dn,
                                    preferred_element_type=jnp.float32)
        m_rep = jax.lax.dot_general(mp16, t_ref[...], dn,
                                    preferred_element_type=jnp.float32)
        z = (e_rep * m_rep).astype(jnp.bfloat16)
        m = jax.lax.dot_general(z, w4_ref[...], dn,
                                preferred_element_type=jnp.float32)
        m_ref[...] = m + jax.lax.dot_general(
            mp, b4_ref[...], dn, preferred_element_type=jnp.float32)
        r_ref[...] = x_ref[...] @ root_ref[...] + b_ref[...]

    return pl.pallas_call(
        body,
        grid=(GRID,),
        in_specs=[
            pl.BlockSpec((TQ, 4 * D), lambda i: (i, 0)),
            pl.BlockSpec((TQ, 4 * F), lambda i: (i, 0)),
            pl.BlockSpec((4 * D, 4 * D * F), lambda i: (0, 0)),
            pl.BlockSpec((4 * F, 4 * D * F), lambda i: (0, 0)),
            pl.BlockSpec((4 * D * F, 4 * F), lambda i: (0, 0)),
            pl.BlockSpec((4 * F, 4 * F), lambda i: (0, 0)),
            pl.BlockSpec((NB, F), lambda i: (i, 0)),
            pl.BlockSpec((F, F), lambda i: (0, 0)),
            pl.BlockSpec((1, F), lambda i: (0, 0)),
        ],
        out_specs=[
            pl.BlockSpec((TQ, 4 * F), lambda i: (i, 0)),
            pl.BlockSpec((NB, F), lambda i: (i, 0)),
        ],
        out_shape=[
            jax.ShapeDtypeStruct((EQ, 4 * F), jnp.float32),
            jax.ShapeDtypeStruct((NP, F), jnp.float32),
        ],
    )(e4, msgs_p, S4, T4, W4, B4, xh, root, bias1r)


def _tc_relu3(p, r):
    def body(p_ref, r_ref, h_ref):
        h_ref[...] = jnp.maximum(p_ref[0] + p_ref[1] + r_ref[...], 0.0)

    return pl.pallas_call(
        body,
        out_shape=jax.ShapeDtypeStruct((NP, F), jnp.float32),
    )(p, r)


def _tc_final(pa, pb, r, dw, db):
    def body(a_ref, b_ref, r_ref, w_ref, db_ref, o_ref):
        h = jnp.maximum(a_ref[...] + b_ref[...] + r_ref[...], 0.0)
        pooled = jnp.sum(h, axis=0, keepdims=True)
        o_ref[...] = pooled @ w_ref[...] + db_ref[...]

    return pl.pallas_call(
        body,
        out_shape=jax.ShapeDtypeStruct((1, 1), jnp.float32),
    )(pa, pb, r, dw, db.reshape(1, 1))


def _expand_w(w_k, b_k):
    Wt = w_k.reshape(D * F, F).astype(jnp.bfloat16)
    Bm = b_k.reshape(F, F)
    eye4b = jnp.eye(4, dtype=jnp.bfloat16)
    eye4f = jnp.eye(4, dtype=jnp.float32)
    W4 = (eye4b[:, None, :, None] * Wt[None, :, None, :]
          ).reshape(4 * D * F, 4 * F)
    B4 = (eye4f[:, None, :, None] * Bm[None, :, None, :]
          ).reshape(4 * F, 4 * F)
    return W4, B4


def kernel(x, edge_index, e, w_k1, b_k1, root1, bias1,
           w_k2, b_k2, root2, bias2, dense_w, dense_b):
    src = edge_index[0]
    tgt = edge_index[1]
    src3 = jnp.concatenate(
        [src, jnp.zeros((EP - E,), jnp.int32)]).reshape(NW, NCH, 128)
    tgt3 = jnp.concatenate(
        [tgt, jnp.full((EP - E,), N, jnp.int32)]).reshape(NW, NCH, 128)
    e4 = jnp.concatenate(
        [e, jnp.zeros((EP - E, D), jnp.float32)]).reshape(EQ, 4 * D)
    x_p = jnp.concatenate([x, jnp.zeros((NP - N, F), jnp.float32)])
    W41, B41 = _expand_w(w_k1, b_k1)
    W42, B42 = _expand_w(w_k2, b_k2)
    zeros_np = jnp.zeros((NP, F), jnp.float32)

    msgs1 = _sc_gather(x, src3)
    m1, r1 = _tc_messages(e4, msgs1.reshape(EQ, 4 * F), W41, B41, x_p,
                          root1, bias1.reshape(1, F))
    p1 = _sc_scatter(m1.reshape(EP, F), tgt3, zeros_np)
    h1 = _tc_relu3(p1, r1)
    msgs2 = _sc_gather(h1, src3)
    m2, r2 = _tc_messages(e4, msgs2.reshape(EQ, 4 * F), W42, B42, h1,
                          root2, bias2.reshape(1, F))
    p2 = _sc_scatter(m2.reshape(EP, F), tgt3, zeros_np)
    return _tc_final(p2[0, :N], p2[1, :N], r2[:N], dense_w, dense_b)


# speedup vs baseline: 3.9273x; 1.1709x over previous
import functools

import jax
import jax.numpy as jnp
from jax import lax
from jax.experimental import pallas as pl
from jax.experimental.pallas import tpu as pltpu
from jax.experimental.pallas import tpu_sc as plsc

N = 10000
E = 80000
F = 32
D = 16

NC, NS = 2, 16
NW = NC * NS
EPW = 2560
NCH = 20
NP = 10240
RPT = NP // NS
EQ = E // 4

_MESH = dict(core_axis_name="c", subcore_axis_name="s")
_SC_PARAMS = pltpu.CompilerParams(use_tc_tiling_on_sc=False)


def _sc_gather(table, idx3):
    @functools.partial(
        pl.kernel,
        out_type=jax.ShapeDtypeStruct((E, F), jnp.float32),
        mesh=plsc.VectorSubcoreMesh(**_MESH),
        scratch_types=[
            pltpu.VMEM((NCH, 128), jnp.int32),
            pltpu.VMEM((EPW, F), jnp.float32),
            pltpu.SemaphoreType.DMA,
        ],
        compiler_params=_SC_PARAMS,
    )
    def k(table_hbm, idx_hbm, out_hbm, idx_v, rows_v, sem):
        cid = lax.axis_index("c")
        sid = lax.axis_index("s")
        wid = sid * NC + cid
        pltpu.sync_copy(idx_hbm.at[wid], idx_v)
        head = [
            pltpu.async_copy(table_hbm.at[idx_v.at[j]],
                             rows_v.at[pl.ds(j * 128, 128)], sem)
            for j in range(5)
        ]

        @pl.when(wid < NW - 1)
        def _tail_gather():
            tail = [
                pltpu.async_copy(table_hbm.at[idx_v.at[j]],
                                 rows_v.at[pl.ds(j * 128, 128)], sem)
                for j in range(5, NCH)
            ]
            for c in tail:
                c.wait()

        for c in head:
            c.wait()
        pltpu.sync_copy(rows_v.at[pl.ds(0, 640)],
                        out_hbm.at[pl.ds(wid * EPW, 640)])

        @pl.when(wid < NW - 1)
        def _tail_out():
            pltpu.sync_copy(rows_v.at[pl.ds(640, 1920)],
                            out_hbm.at[pl.ds(wid * EPW + 640, 1920)])

    return k(table, idx3)


def _sc_scatter(m, tgt3, zeros_np):
    @functools.partial(
        pl.kernel,
        out_type=jax.ShapeDtypeStruct((NC, NP, F), jnp.float32),
        mesh=plsc.VectorSubcoreMesh(**_MESH),
        scratch_types=[
            pltpu.VMEM((NCH, 128), jnp.int32),
            pltpu.VMEM((EPW, F), jnp.float32),
            pltpu.VMEM((RPT, F), jnp.float32),
            pltpu.VMEM_SHARED((NP, F), jnp.float32),
            pltpu.SemaphoreType.DMA,
        ],
        compiler_params=_SC_PARAMS,
    )
    def k(m_hbm, tgt_hbm, z_hbm, p_hbm, idx_v, m_v, stage_v, acc_sh, sem):
        cid = lax.axis_index("c")
        sid = lax.axis_index("s")
        wid = sid * NC + cid
        pltpu.sync_copy(z_hbm.at[pl.ds(sid * RPT, RPT)], stage_v)
        pltpu.sync_copy(stage_v, acc_sh.at[pl.ds(sid * RPT, RPT)])
        pltpu.sync_copy(tgt_hbm.at[wid], idx_v)
        pltpu.sync_copy(m_hbm.at[pl.ds(wid * EPW, 640)],
                        m_v.at[pl.ds(0, 640)])

        @pl.when(wid < NW - 1)
        def _tail_in():
            pltpu.sync_copy(m_hbm.at[pl.ds(wid * EPW + 640, 1920)],
                            m_v.at[pl.ds(640, 1920)])

        plsc.subcore_barrier()
        head = [
            pltpu.async_copy(m_v.at[pl.ds(j * 128, 128)],
                             acc_sh.at[idx_v.at[j]], sem, add=True)
            for j in range(5)
        ]

        @pl.when(wid < NW - 1)
        def _tail_add():
            tail = [
                pltpu.async_copy(m_v.at[pl.ds(j * 128, 128)],
                                 acc_sh.at[idx_v.at[j]], sem, add=True)
                for j in range(5, NCH)
            ]
            for c in tail:
                c.wait()

        for c in head:
            c.wait()
        plsc.subcore_barrier()
        pltpu.sync_copy(acc_sh.at[pl.ds(sid * RPT, RPT)], stage_v)
        pltpu.sync_copy(stage_v, p_hbm.at[cid, pl.ds(sid * RPT, RPT)])

    return k(m, tgt3, zeros_np)


def _tc_messages(e4, msgs_p, W4, B4, xh, root, bias1r):
    GRID = 125
    TQ = EQ // GRID
    NB = N // GRID

    v = jnp.arange(4 * D * F)
    jj = v // (D * F)
    dd = (v % (D * F)) // F
    bb = v % F
    S4 = (jnp.arange(4 * D)[:, None] == (jj * D + dd)[None, :]
          ).astype(jnp.bfloat16)
    T4 = (jnp.arange(4 * F)[:, None] == (jj * F + bb)[None, :]
          ).astype(jnp.bfloat16)
    dn = (((1,), (0,)), ((), ()))

    def body(e_ref, mg_ref, s_ref, t_ref, w4_ref, b4_ref, x_ref, root_ref,
             b_ref, m_ref, r_ref):
        eb = e_ref[...].astype(jnp.bfloat16)
        mp = mg_ref[...]
        mp16 = mp.astype(jnp.bfloat16)
        e_rep = jax.lax.dot_general(eb, s_ref[...], dn,
                                    preferred_element_type=jnp.float32)
        m_rep = jax.lax.dot_general(mp16, t_ref[...], dn,
                                    preferred_element_type=jnp.float32)
        z = (e_rep * m_rep).astype(jnp.bfloat16)
        m = jax.lax.dot_general(z, w4_ref[...], dn,
                                preferred_element_type=jnp.float32)
        m_ref[...] = m + jax.lax.dot_general(
            mp, b4_ref[...], dn, preferred_element_type=jnp.float32)
        r_ref[...] = x_ref[...] @ root_ref[...] + b_ref[...]

    return pl.pallas_call(
        body,
        grid=(GRID,),
        in_specs=[
            pl.BlockSpec((TQ, 4 * D), lambda i: (i, 0)),
            pl.BlockSpec((TQ, 4 * F), lambda i: (i, 0)),
            pl.BlockSpec((4 * D, 4 * D * F), lambda i: (0, 0)),
            pl.BlockSpec((4 * F, 4 * D * F), lambda i: (0, 0)),
            pl.BlockSpec((4 * D * F, 4 * F), lambda i: (0, 0)),
            pl.BlockSpec((4 * F, 4 * F), lambda i: (0, 0)),
            pl.BlockSpec((NB, F), lambda i: (i, 0)),
            pl.BlockSpec((F, F), lambda i: (0, 0)),
            pl.BlockSpec((1, F), lambda i: (0, 0)),
        ],
        out_specs=[
            pl.BlockSpec((TQ, 4 * F), lambda i: (i, 0)),
            pl.BlockSpec((NB, F), lambda i: (i, 0)),
        ],
        out_shape=[
            jax.ShapeDtypeStruct((EQ, 4 * F), jnp.float32),
            jax.ShapeDtypeStruct((N, F), jnp.float32),
        ],
    )(e4, msgs_p, S4, T4, W4, B4, xh, root, bias1r)


def _tc_relu3(pa, pb, r):
    def body(a_ref, b_ref, r_ref, h_ref):
        h_ref[...] = jnp.maximum(a_ref[...] + b_ref[...] + r_ref[...], 0.0)

    return pl.pallas_call(
        body,
        out_shape=jax.ShapeDtypeStruct((N, F), jnp.float32),
    )(pa, pb, r)


def _tc_final(pa, pb, r, dw, db):
    def body(a_ref, b_ref, r_ref, w_ref, db_ref, o_ref):
        h = jnp.maximum(a_ref[...] + b_ref[...] + r_ref[...], 0.0)
        pooled = jnp.sum(h, axis=0, keepdims=True)
        o_ref[...] = pooled @ w_ref[...] + db_ref[...]

    return pl.pallas_call(
        body,
        out_shape=jax.ShapeDtypeStruct((1, 1), jnp.float32),
    )(pa, pb, r, dw, db.reshape(1, 1))


def _expand_w(w_k, b_k):
    Wt = w_k.reshape(D * F, F).astype(jnp.bfloat16)
    Bm = b_k.reshape(F, F)
    eye4b = jnp.eye(4, dtype=jnp.bfloat16)
    eye4f = jnp.eye(4, dtype=jnp.float32)
    W4 = (eye4b[:, None, :, None] * Wt[None, :, None, :]
          ).reshape(4 * D * F, 4 * F)
    B4 = (eye4f[:, None, :, None] * Bm[None, :, None, :]
          ).reshape(4 * F, 4 * F)
    return W4, B4


def kernel(x, edge_index, e, w_k1, b_k1, root1, bias1,
           w_k2, b_k2, root2, bias2, dense_w, dense_b):
    src = edge_index[0]
    tgt = edge_index[1]
    pad = jnp.zeros((NW * NCH * 128 - E,), jnp.int32)
    src3 = jnp.concatenate([src, pad]).reshape(NW, NCH, 128)
    tgt3 = jnp.concatenate([tgt, pad]).reshape(NW, NCH, 128)
    e4 = e.reshape(EQ, 4 * D)
    W41, B41 = _expand_w(w_k1, b_k1)
    W42, B42 = _expand_w(w_k2, b_k2)
    zeros_np = jnp.zeros((NP, F), jnp.float32)

    msgs1 = _sc_gather(x, src3)
    m1, r1 = _tc_messages(e4, msgs1.reshape(EQ, 4 * F), W41, B41, x,
                          root1, bias1.reshape(1, F))
    p1 = _sc_scatter(m1.reshape(E, F), tgt3, zeros_np)
    h1 = _tc_relu3(p1[0, :N], p1[1, :N], r1)
    msgs2 = _sc_gather(h1, src3)
    m2, r2 = _tc_messages(e4, msgs2.reshape(EQ, 4 * F), W42, B42, h1,
                          root2, bias2.reshape(1, F))
    p2 = _sc_scatter(m2.reshape(E, F), tgt3, zeros_np)
    return _tc_final(p2[0, :N], p2[1, :N], r2[:N], dense_w, dense_b)


# speedup vs baseline: 4.2920x; 1.0929x over previous
import functools

import jax
import jax.numpy as jnp
from jax import lax
from jax.experimental import pallas as pl
from jax.experimental.pallas import tpu as pltpu
from jax.experimental.pallas import tpu_sc as plsc

N = 10000
E = 80000
F = 32
D = 16

NC, NS = 2, 16
NW = NC * NS
EPW = 2560
NCH = 20
NP = 10240
RPT = NP // NS
EQ = E // 4

_MESH = dict(core_axis_name="c", subcore_axis_name="s")
_SC_PARAMS = pltpu.CompilerParams(use_tc_tiling_on_sc=False)


def _sc_gather(table, idx3):
    @functools.partial(
        pl.kernel,
        out_type=jax.ShapeDtypeStruct((E, F), jnp.float32),
        mesh=plsc.VectorSubcoreMesh(**_MESH),
        scratch_types=[
            pltpu.VMEM((NCH, 128), jnp.int32),
            pltpu.VMEM((EPW, F), jnp.float32),
            pltpu.SemaphoreType.DMA,
        ],
        compiler_params=_SC_PARAMS,
    )
    def k(table_hbm, idx_hbm, out_hbm, idx_v, rows_v, sem):
        cid = lax.axis_index("c")
        sid = lax.axis_index("s")
        wid = sid * NC + cid
        pltpu.sync_copy(idx_hbm.at[wid], idx_v)
        head = [
            pltpu.async_copy(table_hbm.at[idx_v.at[j]],
                             rows_v.at[pl.ds(j * 128, 128)], sem)
            for j in range(5)
        ]

        @pl.when(wid < NW - 1)
        def _tail_gather():
            tail = [
                pltpu.async_copy(table_hbm.at[idx_v.at[j]],
                                 rows_v.at[pl.ds(j * 128, 128)], sem)
                for j in range(5, NCH)
            ]
            for c in tail:
                c.wait()

        for c in head:
            c.wait()
        pltpu.sync_copy(rows_v.at[pl.ds(0, 640)],
                        out_hbm.at[pl.ds(wid * EPW, 640)])

        @pl.when(wid < NW - 1)
        def _tail_out():
            pltpu.sync_copy(rows_v.at[pl.ds(640, 1920)],
                            out_hbm.at[pl.ds(wid * EPW + 640, 1920)])

    return k(table, idx3)


def _sc_scatter(m, tgt3, zeros_np):
    @functools.partial(
        pl.kernel,
        out_type=jax.ShapeDtypeStruct((NC, NP, F), jnp.float32),
        mesh=plsc.VectorSubcoreMesh(**_MESH),
        scratch_types=[
            pltpu.VMEM((NCH, 128), jnp.int32),
            pltpu.VMEM((EPW, F), jnp.float32),
            pltpu.VMEM((RPT, F), jnp.float32),
            pltpu.VMEM_SHARED((NP, F), jnp.float32),
            pltpu.SemaphoreType.DMA,
        ],
        compiler_params=_SC_PARAMS,
    )
    def k(m_hbm, tgt_hbm, z_hbm, p_hbm, idx_v, m_v, stage_v, acc_sh, sem):
        cid = lax.axis_index("c")
        sid = lax.axis_index("s")
        wid = sid * NC + cid
        pltpu.sync_copy(z_hbm.at[pl.ds(sid * RPT, RPT)], stage_v)
        pltpu.sync_copy(stage_v, acc_sh.at[pl.ds(sid * RPT, RPT)])
        pltpu.sync_copy(tgt_hbm.at[wid], idx_v)
        pltpu.sync_copy(m_hbm.at[pl.ds(wid * EPW, 640)],
                        m_v.at[pl.ds(0, 640)])

        @pl.when(wid < NW - 1)
        def _tail_in():
            pltpu.sync_copy(m_hbm.at[pl.ds(wid * EPW + 640, 1920)],
                            m_v.at[pl.ds(640, 1920)])

        plsc.subcore_barrier()
        head = [
            pltpu.async_copy(m_v.at[pl.ds(j * 128, 128)],
                             acc_sh.at[idx_v.at[j]], sem, add=True)
            for j in range(5)
        ]

        @pl.when(wid < NW - 1)
        def _tail_add():
            tail = [
                pltpu.async_copy(m_v.at[pl.ds(j * 128, 128)],
                                 acc_sh.at[idx_v.at[j]], sem, add=True)
                for j in range(5, NCH)
            ]
            for c in tail:
                c.wait()

        for c in head:
            c.wait()
        plsc.subcore_barrier()
        pltpu.sync_copy(acc_sh.at[pl.ds(sid * RPT, RPT)], stage_v)
        pltpu.sync_copy(stage_v, p_hbm.at[cid, pl.ds(sid * RPT, RPT)])

    return k(m, tgt3, zeros_np)


def _tc_messages(e4, msgs_p, W4, B4, xh, root, bias1r):
    GRID = 50
    TQ = EQ // GRID
    NB = N // GRID

    v = jnp.arange(4 * D * F)
    jj = v // (D * F)
    dd = (v % (D * F)) // F
    bb = v % F
    S4 = (jnp.arange(4 * D)[:, None] == (jj * D + dd)[None, :]
          ).astype(jnp.bfloat16)
    T4 = (jnp.arange(4 * F)[:, None] == (jj * F + bb)[None, :]
          ).astype(jnp.bfloat16)
    dn = (((1,), (0,)), ((), ()))

    def body(e_ref, mg_ref, s_ref, t_ref, w4_ref, b4_ref, x_ref, root_ref,
             b_ref, m_ref, r_ref):
        eb = e_ref[...].astype(jnp.bfloat16)
        mp = mg_ref[...]
        mp16 = mp.astype(jnp.bfloat16)
        e_rep = jax.lax.dot_general(eb, s_ref[...], dn,
                                    preferred_element_type=jnp.float32)
        m_rep = jax.lax.dot_general(mp16, t_ref[...], dn,
                                    preferred_element_type=jnp.float32)
        z = (e_rep * m_rep).astype(jnp.bfloat16)
        m = jax.lax.dot_general(z, w4_ref[...], dn,
                                preferred_element_type=jnp.float32)
        m_ref[...] = m + jax.lax.dot_general(
            mp, b4_ref[...], dn, preferred_element_type=jnp.float32)
        r_ref[...] = x_ref[...] @ root_ref[...] + b_ref[...]

    return pl.pallas_call(
        body,
        grid=(GRID,),
        in_specs=[
            pl.BlockSpec((TQ, 4 * D), lambda i: (i, 0)),
            pl.BlockSpec((TQ, 4 * F), lambda i: (i, 0)),
            pl.BlockSpec((4 * D, 4 * D * F), lambda i: (0, 0)),
            pl.BlockSpec((4 * F, 4 * D * F), lambda i: (0, 0)),
            pl.BlockSpec((4 * D * F, 4 * F), lambda i: (0, 0)),
            pl.BlockSpec((4 * F, 4 * F), lambda i: (0, 0)),
            pl.BlockSpec((NB, F), lambda i: (i, 0)),
            pl.BlockSpec((F, F), lambda i: (0, 0)),
            pl.BlockSpec((1, F), lambda i: (0, 0)),
        ],
        out_specs=[
            pl.BlockSpec((TQ, 4 * F), lambda i: (i, 0)),
            pl.BlockSpec((NB, F), lambda i: (i, 0)),
        ],
        out_shape=[
            jax.ShapeDtypeStruct((EQ, 4 * F), jnp.float32),
            jax.ShapeDtypeStruct((N, F), jnp.float32),
        ],
    )(e4, msgs_p, S4, T4, W4, B4, xh, root, bias1r)


def _tc_relu3(pa, pb, r):
    def body(a_ref, b_ref, r_ref, h_ref):
        h_ref[...] = jnp.maximum(a_ref[...] + b_ref[...] + r_ref[...], 0.0)

    return pl.pallas_call(
        body,
        out_shape=jax.ShapeDtypeStruct((N, F), jnp.float32),
    )(pa, pb, r)


def _tc_final(pa, pb, r, dw, db):
    def body(a_ref, b_ref, r_ref, w_ref, db_ref, o_ref):
        h = jnp.maximum(a_ref[...] + b_ref[...] + r_ref[...], 0.0)
        pooled = jnp.sum(h, axis=0, keepdims=True)
        o_ref[...] = pooled @ w_ref[...] + db_ref[...]

    return pl.pallas_call(
        body,
        out_shape=jax.ShapeDtypeStruct((1, 1), jnp.float32),
    )(pa, pb, r, dw, db.reshape(1, 1))


def _expand_w(w_k, b_k):
    Wt = w_k.reshape(D * F, F).astype(jnp.bfloat16)
    Bm = b_k.reshape(F, F)
    eye4b = jnp.eye(4, dtype=jnp.bfloat16)
    eye4f = jnp.eye(4, dtype=jnp.float32)
    W4 = (eye4b[:, None, :, None] * Wt[None, :, None, :]
          ).reshape(4 * D * F, 4 * F)
    B4 = (eye4f[:, None, :, None] * Bm[None, :, None, :]
          ).reshape(4 * F, 4 * F)
    return W4, B4


def kernel(x, edge_index, e, w_k1, b_k1, root1, bias1,
           w_k2, b_k2, root2, bias2, dense_w, dense_b):
    src = edge_index[0]
    tgt = edge_index[1]
    pad = jnp.zeros((NW * NCH * 128 - E,), jnp.int32)
    src3 = jnp.concatenate([src, pad]).reshape(NW, NCH, 128)
    tgt3 = jnp.concatenate([tgt, pad]).reshape(NW, NCH, 128)
    e4 = e.reshape(EQ, 4 * D)
    W41, B41 = _expand_w(w_k1, b_k1)
    W42, B42 = _expand_w(w_k2, b_k2)
    zeros_np = jnp.zeros((NP, F), jnp.float32)

    msgs1 = _sc_gather(x, src3)
    m1, r1 = _tc_messages(e4, msgs1.reshape(EQ, 4 * F), W41, B41, x,
                          root1, bias1.reshape(1, F))
    p1 = _sc_scatter(m1.reshape(E, F), tgt3, zeros_np)
    h1 = _tc_relu3(p1[0, :N], p1[1, :N], r1)
    msgs2 = _sc_gather(h1, src3)
    m2, r2 = _tc_messages(e4, msgs2.reshape(EQ, 4 * F), W42, B42, h1,
                          root2, bias2.reshape(1, F))
    p2 = _sc_scatter(m2.reshape(E, F), tgt3, zeros_np)
    return _tc_final(p2[0, :N], p2[1, :N], r2[:N], dense_w, dense_b)


# speedup vs baseline: 4.4659x; 1.0405x over previous
import functools

import jax
import jax.numpy as jnp
from jax import lax
from jax.experimental import pallas as pl
from jax.experimental.pallas import tpu as pltpu
from jax.experimental.pallas import tpu_sc as plsc

N = 10000
E = 80000
F = 32
D = 16

NC, NS = 2, 16
NW = NC * NS
EPW = 2560
NCH = 20
NP = 10240
RPT = NP // NS
EQ = E // 4

_MESH = dict(core_axis_name="c", subcore_axis_name="s")
_SC_PARAMS = pltpu.CompilerParams(use_tc_tiling_on_sc=False)


def _sc_gather(table, idx3):
    @functools.partial(
        pl.kernel,
        out_type=jax.ShapeDtypeStruct((E, F), jnp.float32),
        mesh=plsc.VectorSubcoreMesh(**_MESH),
        scratch_types=[
            pltpu.VMEM((NCH, 128), jnp.int32),
            pltpu.VMEM((EPW, F), jnp.float32),
            pltpu.SemaphoreType.DMA,
        ],
        compiler_params=_SC_PARAMS,
    )
    def k(table_hbm, idx_hbm, out_hbm, idx_v, rows_v, sem):
        cid = lax.axis_index("c")
        sid = lax.axis_index("s")
        wid = sid * NC + cid
        pltpu.sync_copy(idx_hbm.at[wid], idx_v)
        head = [
            pltpu.async_copy(table_hbm.at[idx_v.at[j]],
                             rows_v.at[pl.ds(j * 128, 128)], sem)
            for j in range(5)
        ]

        @pl.when(wid < NW - 1)
        def _tail_gather():
            tail = [
                pltpu.async_copy(table_hbm.at[idx_v.at[j]],
                                 rows_v.at[pl.ds(j * 128, 128)], sem)
                for j in range(5, NCH)
            ]
            for c in tail:
                c.wait()

        for c in head:
            c.wait()
        pltpu.sync_copy(rows_v.at[pl.ds(0, 640)],
                        out_hbm.at[pl.ds(wid * EPW, 640)])

        @pl.when(wid < NW - 1)
        def _tail_out():
            pltpu.sync_copy(rows_v.at[pl.ds(640, 1920)],
                            out_hbm.at[pl.ds(wid * EPW + 640, 1920)])

    return k(table, idx3)


def _sc_scatter(m, tgt3, zeros_np):
    @functools.partial(
        pl.kernel,
        out_type=jax.ShapeDtypeStruct((NC, NP, F), jnp.float32),
        mesh=plsc.VectorSubcoreMesh(**_MESH),
        scratch_types=[
            pltpu.VMEM((NCH, 128), jnp.int32),
            pltpu.VMEM((EPW, F), jnp.float32),
            pltpu.VMEM((RPT, F), jnp.float32),
            pltpu.VMEM_SHARED((NP, F), jnp.float32),
            pltpu.SemaphoreType.DMA,
        ],
        compiler_params=_SC_PARAMS,
    )
    def k(m_hbm, tgt_hbm, z_hbm, p_hbm, idx_v, m_v, stage_v, acc_sh, sem):
        cid = lax.axis_index("c")
        sid = lax.axis_index("s")
        wid = sid * NC + cid
        pltpu.sync_copy(z_hbm.at[pl.ds(sid * RPT, RPT)], stage_v)
        pltpu.sync_copy(stage_v, acc_sh.at[pl.ds(sid * RPT, RPT)])
        pltpu.sync_copy(tgt_hbm.at[wid], idx_v)
        pltpu.sync_copy(m_hbm.at[pl.ds(wid * EPW, 640)],
                        m_v.at[pl.ds(0, 640)])

        @pl.when(wid < NW - 1)
        def _tail_in():
            pltpu.sync_copy(m_hbm.at[pl.ds(wid * EPW + 640, 1920)],
                            m_v.at[pl.ds(640, 1920)])

        plsc.subcore_barrier()
        head = [
            pltpu.async_copy(m_v.at[pl.ds(j * 128, 128)],
                             acc_sh.at[idx_v.at[j]], sem, add=True)
            for j in range(5)
        ]

        @pl.when(wid < NW - 1)
        def _tail_add():
            tail = [
                pltpu.async_copy(m_v.at[pl.ds(j * 128, 128)],
                                 acc_sh.at[idx_v.at[j]], sem, add=True)
                for j in range(5, NCH)
            ]
            for c in tail:
                c.wait()

        for c in head:
            c.wait()
        plsc.subcore_barrier()
        pltpu.sync_copy(acc_sh.at[pl.ds(sid * RPT, RPT)], stage_v)
        pltpu.sync_copy(stage_v, p_hbm.at[cid, pl.ds(sid * RPT, RPT)])

    return k(m, tgt3, zeros_np)


def _tc_messages(e4, msgs_p, W4, B4, xh, root, bias1r):
    GRID = 25
    TQ = EQ // GRID
    NB = N // GRID

    v = jnp.arange(4 * D * F)
    jj = v // (D * F)
    dd = (v % (D * F)) // F
    bb = v % F
    S4 = (jnp.arange(4 * D)[:, None] == (jj * D + dd)[None, :]
          ).astype(jnp.bfloat16)
    T4 = (jnp.arange(4 * F)[:, None] == (jj * F + bb)[None, :]
          ).astype(jnp.bfloat16)
    dn = (((1,), (0,)), ((), ()))

    def body(e_ref, mg_ref, s_ref, t_ref, w4_ref, b4_ref, x_ref, root_ref,
             b_ref, m_ref, r_ref):
        eb = e_ref[...].astype(jnp.bfloat16)
        mp = mg_ref[...]
        mp16 = mp.astype(jnp.bfloat16)
        e_rep = jax.lax.dot_general(eb, s_ref[...], dn,
                                    preferred_element_type=jnp.float32)
        m_rep = jax.lax.dot_general(mp16, t_ref[...], dn,
                                    preferred_element_type=jnp.float32)
        z = (e_rep * m_rep).astype(jnp.bfloat16)
        m = jax.lax.dot_general(z, w4_ref[...], dn,
                                preferred_element_type=jnp.float32)
        m_ref[...] = m + jax.lax.dot_general(
            mp, b4_ref[...], dn, preferred_element_type=jnp.float32)
        r_ref[...] = x_ref[...] @ root_ref[...] + b_ref[...]

    return pl.pallas_call(
        body,
        grid=(GRID,),
        in_specs=[
            pl.BlockSpec((TQ, 4 * D), lambda i: (i, 0)),
            pl.BlockSpec((TQ, 4 * F), lambda i: (i, 0)),
            pl.BlockSpec((4 * D, 4 * D * F), lambda i: (0, 0)),
            pl.BlockSpec((4 * F, 4 * D * F), lambda i: (0, 0)),
            pl.BlockSpec((4 * D * F, 4 * F), lambda i: (0, 0)),
            pl.BlockSpec((4 * F, 4 * F), lambda i: (0, 0)),
            pl.BlockSpec((NB, F), lambda i: (i, 0)),
            pl.BlockSpec((F, F), lambda i: (0, 0)),
            pl.BlockSpec((1, F), lambda i: (0, 0)),
        ],
        out_specs=[
            pl.BlockSpec((TQ, 4 * F), lambda i: (i, 0)),
            pl.BlockSpec((NB, F), lambda i: (i, 0)),
        ],
        out_shape=[
            jax.ShapeDtypeStruct((EQ, 4 * F), jnp.float32),
            jax.ShapeDtypeStruct((N, F), jnp.float32),
        ],
    )(e4, msgs_p, S4, T4, W4, B4, xh, root, bias1r)


def _tc_relu3(pa, pb, r):
    def body(a_ref, b_ref, r_ref, h_ref):
        h_ref[...] = jnp.maximum(a_ref[...] + b_ref[...] + r_ref[...], 0.0)

    return pl.pallas_call(
        body,
        out_shape=jax.ShapeDtypeStruct((N, F), jnp.float32),
    )(pa, pb, r)


def _tc_final(pa, pb, r, dw, db):
    def body(a_ref, b_ref, r_ref, w_ref, db_ref, o_ref):
        h = jnp.maximum(a_ref[...] + b_ref[...] + r_ref[...], 0.0)
        pooled = jnp.sum(h, axis=0, keepdims=True)
        o_ref[...] = pooled @ w_ref[...] + db_ref[...]

    return pl.pallas_call(
        body,
        out_shape=jax.ShapeDtypeStruct((1, 1), jnp.float32),
    )(pa, pb, r, dw, db.reshape(1, 1))


def _expand_w(w_k, b_k):
    Wt = w_k.reshape(D * F, F).astype(jnp.bfloat16)
    Bm = b_k.reshape(F, F)
    eye4b = jnp.eye(4, dtype=jnp.bfloat16)
    eye4f = jnp.eye(4, dtype=jnp.float32)
    W4 = (eye4b[:, None, :, None] * Wt[None, :, None, :]
          ).reshape(4 * D * F, 4 * F)
    B4 = (eye4f[:, None, :, None] * Bm[None, :, None, :]
          ).reshape(4 * F, 4 * F)
    return W4, B4


def kernel(x, edge_index, e, w_k1, b_k1, root1, bias1,
           w_k2, b_k2, root2, bias2, dense_w, dense_b):
    src = edge_index[0]
    tgt = edge_index[1]
    pad = jnp.zeros((NW * NCH * 128 - E,), jnp.int32)
    src3 = jnp.concatenate([src, pad]).reshape(NW, NCH, 128)
    tgt3 = jnp.concatenate([tgt, pad]).reshape(NW, NCH, 128)
    e4 = e.reshape(EQ, 4 * D)
    W41, B41 = _expand_w(w_k1, b_k1)
    W42, B42 = _expand_w(w_k2, b_k2)
    zeros_np = jnp.zeros((NP, F), jnp.float32)

    msgs1 = _sc_gather(x, src3)
    m1, r1 = _tc_messages(e4, msgs1.reshape(EQ, 4 * F), W41, B41, x,
                          root1, bias1.reshape(1, F))
    p1 = _sc_scatter(m1.reshape(E, F), tgt3, zeros_np)
    h1 = _tc_relu3(p1[0, :N], p1[1, :N], r1)
    msgs2 = _sc_gather(h1, src3)
    m2, r2 = _tc_messages(e4, msgs2.reshape(EQ, 4 * F), W42, B42, h1,
                          root2, bias2.reshape(1, F))
    p2 = _sc_scatter(m2.reshape(E, F), tgt3, zeros_np)
    return _tc_final(p2[0, :N], p2[1, :N], r2[:N], dense_w, dense_b)
